# baseline (speedup 1.0000x reference)
import functools

import jax
import jax.numpy as jnp
from jax import lax
from jax.experimental import pallas as pl
from jax.experimental.pallas import tpu as pltpu
from jax.experimental.pallas import tpu_sc as plsc

T, B = 32, 64
P, I, R = 1024, 64, 128
H, N = 64, 4096
K = 8

NC, NS, L = 1, 16, 16
NW = NC * NS
RW = B // NW
NCHUNK = N // L


TBLK = 16
NT = T // TBLK


def _scores_body(periph_ref, imu_ref, traj_ref, w1_ref, b1_ref, w2_ref,
                 b2_ref, scores_ref, acc_p, acc_i, acc_t):
    t = pl.program_id(0)

    @pl.when(t == 0)
    def _():
        acc_p[...] = jnp.zeros_like(acc_p)
        acc_i[...] = jnp.zeros_like(acc_i)
        acc_t[...] = jnp.zeros_like(acc_t)

    acc_p[...] += jnp.sum(periph_ref[...], axis=0)
    acc_i[...] += jnp.sum(imu_ref[...], axis=0)
    acc_t[...] += jnp.sum(traj_ref[...], axis=0)

    @pl.when(t == NT - 1)
    def _():
        inv = jnp.float32(1.0 / T)
        xp = acc_p[...] * inv
        xi = acc_i[...] * inv
        xt = acc_t[...] * inv
        w1 = w1_ref[...]
        h = (jnp.dot(xp, w1[0:P], preferred_element_type=jnp.float32)
             + jnp.dot(xi, w1[P:P + I], preferred_element_type=jnp.float32)
             + jnp.dot(xt, w1[P + I:P + I + R],
                       preferred_element_type=jnp.float32)
             + b1_ref[...])
        h = jnp.maximum(h, 0.0)
        scores_ref[...] = (
            jnp.dot(h, w2_ref[...], preferred_element_type=jnp.float32)
            + b2_ref[...])


def _tc_scores(periph_seq, imu_seq, traj_seq, W1, b1, W2, b2):
    return pl.pallas_call(
        _scores_body,
        grid=(NT,),
        in_specs=[
            pl.BlockSpec((TBLK, B, P), lambda t: (t, 0, 0)),
            pl.BlockSpec((TBLK, B, I), lambda t: (t, 0, 0)),
            pl.BlockSpec((TBLK, B, R), lambda t: (t, 0, 0)),
            pl.BlockSpec((P + I + R, H), lambda t: (0, 0)),
            pl.BlockSpec((1, H), lambda t: (0, 0)),
            pl.BlockSpec((H, N), lambda t: (0, 0)),
            pl.BlockSpec((1, N), lambda t: (0, 0)),
        ],
        out_specs=pl.BlockSpec((B, N), lambda t: (0, 0)),
        out_shape=jax.ShapeDtypeStruct((B, N), jnp.float32),
        scratch_shapes=[pltpu.VMEM((B, P), jnp.float32),
                        pltpu.VMEM((B, I), jnp.float32),
                        pltpu.VMEM((B, R), jnp.float32)],
    )(periph_seq, imu_seq, traj_seq, W1, b1.reshape(1, H), W2,
      b2.reshape(1, N))


_GATHER_DNUMS = lax.GatherDimensionNumbers(
    offset_dims=(), collapsed_slice_dims=(0,), start_index_map=(0,))


def _shuffle(x, idx):
    return lax.gather(x, idx, _GATHER_DNUMS, slice_sizes=(1,),
                      mode=lax.GatherScatterMode.PROMISE_IN_BOUNDS)


def _xor_indices():
    iota = lax.iota(jnp.int32, L)
    return [jnp.reshape(jnp.bitwise_xor(iota, s), (L, 1))
            for s in (1, 2, 4, 8)]


def _allmax(x, xor_idx):
    for idx in xor_idx:
        x = jnp.maximum(x, _shuffle(x, idx))
    return x


def _allsum(x, xor_idx):
    for idx in xor_idx:
        x = x + _shuffle(x, idx)
    return x


@functools.partial(
    pl.kernel,
    mesh=plsc.VectorSubcoreMesh(core_axis_name="c", subcore_axis_name="s",
                                num_cores=NC),
    out_type=jax.ShapeDtypeStruct((B, N), jnp.float32),
    scratch_types=[pltpu.VMEM((RW, N), jnp.float32),
                   pltpu.VMEM((RW, N), jnp.float32)],
)
def _sc_gate(scores_hbm, gate_hbm, rows_v, gate_v):
    wid = lax.axis_index("s") * NC + lax.axis_index("c")
    base = wid * RW
    pltpu.sync_copy(scores_hbm.at[pl.ds(base, RW)], rows_v)

    xor_idx = _xor_indices()
    neg = jnp.full((L,), -3.0e38, jnp.float32)
    for r in range(RW):
        def chunk_body(j, lists):
            v = [rows_v[r, pl.ds((j * K + i) * L, L)] for i in range(K)]
            for a, b in ((0, 1), (2, 3), (4, 5), (6, 7),
                         (0, 2), (1, 3), (4, 6), (5, 7),
                         (1, 2), (5, 6), (0, 4), (3, 7),
                         (1, 5), (2, 6),
                         (1, 4), (3, 6),
                         (2, 4), (3, 5),
                         (3, 4)):
                hi = jnp.maximum(v[a], v[b])
                v[b] = jnp.minimum(v[a], v[b])
                v[a] = hi
            lists = list(lists)
            for k in range(K):
                w = v[k]
                for lv in range(k, K):
                    hi = jnp.maximum(lists[lv], w)
                    w = jnp.minimum(lists[lv], w)
                    lists[lv] = hi
            return tuple(lists)

        lists = lax.fori_loop(0, NCHUNK // K, chunk_body, (neg,) * K)

        cur = jnp.full((L,), 3.0e38, jnp.float32)
        remaining = jnp.full((L,), float(K), jnp.float32)
        thresh = neg
        for _ in range(K):
            mv = neg
            for li in lists:
                mv = jnp.maximum(mv, jnp.where(li < cur, li, neg))
            m = _allmax(mv, xor_idx)
            cnt = jnp.zeros((L,), jnp.float32)
            for li in lists:
                cnt = cnt + jnp.where(li == m, 1.0, 0.0)
            cnt = _allsum(cnt, xor_idx)
            tf = jnp.where(remaining > 0, jnp.float32(1.0), jnp.float32(0.0))
            thresh = tf * m + (1.0 - tf) * thresh
            remaining = remaining - tf * cnt
            cur = m

        def gate_body(j, carry):
            for i in range(K):
                off = (j * K + i) * L
                v = rows_v[r, pl.ds(off, L)]
                gate_v[r, pl.ds(off, L)] = jnp.where(
                    v >= thresh, jnp.float32(1.0), jnp.float32(0.0))
            return carry

        lax.fori_loop(0, NCHUNK // K, gate_body, 0)

    pltpu.sync_copy(gate_v, gate_hbm.at[pl.ds(base, RW)])


def kernel(periph_seq, imu_seq, traj_seq, W1, b1, W2, b2):
    scores = _tc_scores(periph_seq, imu_seq, traj_seq, W1, b1, W2, b2)
    gate = _sc_gate(scores)
    return (scores, gate)


# speedup vs baseline: 1.0228x; 1.0228x over previous
import functools

import jax
import jax.numpy as jnp
from jax import lax
from jax.experimental import pallas as pl
from jax.experimental.pallas import tpu as pltpu
from jax.experimental.pallas import tpu_sc as plsc

T, B = 32, 64
P, I, R = 1024, 64, 128
H, N = 64, 4096
K = 8

NC, NS, L = 2, 16, 16
NW = NC * NS
RW = B // NW
NCHUNK = N // L


TBLK = 16
NT = T // TBLK


B2 = B // 2


def _scores_body(periph_ref, imu_ref, traj_ref, w1_ref, b1_ref, w2_ref,
                 b2_ref, scores_ref, acc_p, acc_i, acc_t):
    t = pl.program_id(1)

    @pl.when(t == 0)
    def _():
        acc_p[...] = jnp.zeros_like(acc_p)
        acc_i[...] = jnp.zeros_like(acc_i)
        acc_t[...] = jnp.zeros_like(acc_t)

    acc_p[...] += jnp.sum(periph_ref[...], axis=0)
    acc_i[...] += jnp.sum(imu_ref[...], axis=0)
    acc_t[...] += jnp.sum(traj_ref[...], axis=0)

    @pl.when(t == NT - 1)
    def _():
        inv = jnp.float32(1.0 / T)
        xp = acc_p[...] * inv
        xi = acc_i[...] * inv
        xt = acc_t[...] * inv
        w1 = w1_ref[...]
        h = (jnp.dot(xp, w1[0:P], preferred_element_type=jnp.float32)
             + jnp.dot(xi, w1[P:P + I], preferred_element_type=jnp.float32)
             + jnp.dot(xt, w1[P + I:P + I + R],
                       preferred_element_type=jnp.float32)
             + b1_ref[...])
        h = jnp.maximum(h, 0.0)
        scores_ref[...] = (
            jnp.dot(h, w2_ref[...], preferred_element_type=jnp.float32)
            + b2_ref[...])


def _tc_scores(periph_seq, imu_seq, traj_seq, W1, b1, W2, b2):
    return pl.pallas_call(
        _scores_body,
        grid=(2, NT),
        in_specs=[
            pl.BlockSpec((TBLK, B2, P), lambda b, t: (t, b, 0)),
            pl.BlockSpec((TBLK, B2, I), lambda b, t: (t, b, 0)),
            pl.BlockSpec((TBLK, B2, R), lambda b, t: (t, b, 0)),
            pl.BlockSpec((P + I + R, H), lambda b, t: (0, 0)),
            pl.BlockSpec((1, H), lambda b, t: (0, 0)),
            pl.BlockSpec((H, N), lambda b, t: (0, 0)),
            pl.BlockSpec((1, N), lambda b, t: (0, 0)),
        ],
        out_specs=pl.BlockSpec((B2, N), lambda b, t: (b, 0)),
        out_shape=jax.ShapeDtypeStruct((B, N), jnp.float32),
        scratch_shapes=[pltpu.VMEM((B2, P), jnp.float32),
                        pltpu.VMEM((B2, I), jnp.float32),
                        pltpu.VMEM((B2, R), jnp.float32)],
        compiler_params=pltpu.CompilerParams(
            dimension_semantics=("parallel", "arbitrary")),
    )(periph_seq, imu_seq, traj_seq, W1, b1.reshape(1, H), W2,
      b2.reshape(1, N))


_GATHER_DNUMS = lax.GatherDimensionNumbers(
    offset_dims=(), collapsed_slice_dims=(0,), start_index_map=(0,))


def _shuffle(x, idx):
    return lax.gather(x, idx, _GATHER_DNUMS, slice_sizes=(1,),
                      mode=lax.GatherScatterMode.PROMISE_IN_BOUNDS)


def _xor_indices():
    iota = lax.iota(jnp.int32, L)
    return [jnp.reshape(jnp.bitwise_xor(iota, s), (L, 1))
            for s in (1, 2, 4, 8)]


def _allmax(x, xor_idx):
    for idx in xor_idx:
        x = jnp.maximum(x, _shuffle(x, idx))
    return x


def _allsum(x, xor_idx):
    for idx in xor_idx:
        x = x + _shuffle(x, idx)
    return x


@functools.partial(
    pl.kernel,
    mesh=plsc.VectorSubcoreMesh(core_axis_name="c", subcore_axis_name="s",
                                num_cores=NC),
    out_type=jax.ShapeDtypeStruct((B, N), jnp.float32),
    scratch_types=[pltpu.VMEM((RW, N), jnp.float32),
                   pltpu.VMEM((RW, N), jnp.float32)],
)
def _sc_gate(scores_hbm, gate_hbm, rows_v, gate_v):
    wid = lax.axis_index("s") * NC + lax.axis_index("c")
    base = wid * RW
    pltpu.sync_copy(scores_hbm.at[pl.ds(base, RW)], rows_v)

    xor_idx = _xor_indices()
    neg = jnp.full((L,), -3.0e38, jnp.float32)
    for r in range(RW):
        def chunk_body(j, lists):
            v = [rows_v[r, pl.ds((j * K + i) * L, L)] for i in range(K)]
            for a, b in ((0, 1), (2, 3), (4, 5), (6, 7),
                         (0, 2), (1, 3), (4, 6), (5, 7),
                         (1, 2), (5, 6), (0, 4), (3, 7),
                         (1, 5), (2, 6),
                         (1, 4), (3, 6),
                         (2, 4), (3, 5),
                         (3, 4)):
                hi = jnp.maximum(v[a], v[b])
                v[b] = jnp.minimum(v[a], v[b])
                v[a] = hi
            lists = list(lists)
            for k in range(K):
                w = v[k]
                for lv in range(k, K):
                    hi = jnp.maximum(lists[lv], w)
                    w = jnp.minimum(lists[lv], w)
                    lists[lv] = hi
            return tuple(lists)

        lists = lax.fori_loop(0, NCHUNK // K, chunk_body, (neg,) * K)

        cur = jnp.full((L,), 3.0e38, jnp.float32)
        remaining = jnp.full((L,), float(K), jnp.float32)
        thresh = neg
        for _ in range(K):
            mv = neg
            for li in lists:
                mv = jnp.maximum(mv, jnp.where(li < cur, li, neg))
            m = _allmax(mv, xor_idx)
            cnt = jnp.zeros((L,), jnp.float32)
            for li in lists:
                cnt = cnt + jnp.where(li == m, 1.0, 0.0)
            cnt = _allsum(cnt, xor_idx)
            tf = jnp.where(remaining > 0, jnp.float32(1.0), jnp.float32(0.0))
            thresh = tf * m + (1.0 - tf) * thresh
            remaining = remaining - tf * cnt
            cur = m

        def gate_body(j, carry):
            for i in range(K):
                off = (j * K + i) * L
                v = rows_v[r, pl.ds(off, L)]
                gate_v[r, pl.ds(off, L)] = jnp.where(
                    v >= thresh, jnp.float32(1.0), jnp.float32(0.0))
            return carry

        lax.fori_loop(0, NCHUNK // K, gate_body, 0)

    pltpu.sync_copy(gate_v, gate_hbm.at[pl.ds(base, RW)])


def kernel(periph_seq, imu_seq, traj_seq, W1, b1, W2, b2):
    scores = _tc_scores(periph_seq, imu_seq, traj_seq, W1, b1, W2, b2)
    gate = _sc_gate(scores)
    return (scores, gate)


# speedup vs baseline: 1.0531x; 1.0296x over previous
import functools

import jax
import jax.numpy as jnp
from jax import lax
from jax.experimental import pallas as pl
from jax.experimental.pallas import tpu as pltpu
from jax.experimental.pallas import tpu_sc as plsc

T, B = 32, 64
P, I, R = 1024, 64, 128
H, N = 64, 4096
K = 8

NC, NS, L = 2, 16, 16
NW = NC * NS
RW = B // NW
NCHUNK = N // L


TBLK = 16
NT = T // TBLK


def _scores_body(periph_ref, imu_ref, traj_ref, w1_ref, b1_ref, w2_ref,
                 b2_ref, scores_ref, acc_p, acc_i, acc_t):
    t = pl.program_id(0)

    @pl.when(t == 0)
    def _():
        acc_p[...] = jnp.zeros_like(acc_p)
        acc_i[...] = jnp.zeros_like(acc_i)
        acc_t[...] = jnp.zeros_like(acc_t)

    acc_p[...] += jnp.sum(periph_ref[...], axis=0)
    acc_i[...] += jnp.sum(imu_ref[...], axis=0)
    acc_t[...] += jnp.sum(traj_ref[...], axis=0)

    @pl.when(t == NT - 1)
    def _():
        inv = jnp.float32(1.0 / T)
        xp = acc_p[...] * inv
        xi = acc_i[...] * inv
        xt = acc_t[...] * inv
        w1 = w1_ref[...]
        h = (jnp.dot(xp, w1[0:P], preferred_element_type=jnp.float32)
             + jnp.dot(xi, w1[P:P + I], preferred_element_type=jnp.float32)
             + jnp.dot(xt, w1[P + I:P + I + R],
                       preferred_element_type=jnp.float32)
             + b1_ref[...])
        h = jnp.maximum(h, 0.0)
        scores_ref[...] = (
            jnp.dot(h, w2_ref[...], preferred_element_type=jnp.float32)
            + b2_ref[...])


def _tc_scores(periph_seq, imu_seq, traj_seq, W1, b1, W2, b2):
    return pl.pallas_call(
        _scores_body,
        grid=(NT,),
        in_specs=[
            pl.BlockSpec((TBLK, B, P), lambda t: (t, 0, 0)),
            pl.BlockSpec((TBLK, B, I), lambda t: (t, 0, 0)),
            pl.BlockSpec((TBLK, B, R), lambda t: (t, 0, 0)),
            pl.BlockSpec((P + I + R, H), lambda t: (0, 0)),
            pl.BlockSpec((1, H), lambda t: (0, 0)),
            pl.BlockSpec((H, N), lambda t: (0, 0)),
            pl.BlockSpec((1, N), lambda t: (0, 0)),
        ],
        out_specs=pl.BlockSpec((B, N), lambda t: (0, 0)),
        out_shape=jax.ShapeDtypeStruct((B, N), jnp.float32),
        scratch_shapes=[pltpu.VMEM((B, P), jnp.float32),
                        pltpu.VMEM((B, I), jnp.float32),
                        pltpu.VMEM((B, R), jnp.float32)],
    )(periph_seq, imu_seq, traj_seq, W1, b1.reshape(1, H), W2,
      b2.reshape(1, N))


_GATHER_DNUMS = lax.GatherDimensionNumbers(
    offset_dims=(), collapsed_slice_dims=(0,), start_index_map=(0,))


def _shuffle(x, idx):
    return lax.gather(x, idx, _GATHER_DNUMS, slice_sizes=(1,),
                      mode=lax.GatherScatterMode.PROMISE_IN_BOUNDS)


def _xor_indices():
    iota = lax.iota(jnp.int32, L)
    return [jnp.reshape(jnp.bitwise_xor(iota, s), (L, 1))
            for s in (1, 2, 4, 8)]


def _allmax(x, xor_idx):
    for idx in xor_idx:
        x = jnp.maximum(x, _shuffle(x, idx))
    return x


def _allsum(x, xor_idx):
    for idx in xor_idx:
        x = x + _shuffle(x, idx)
    return x


@functools.partial(
    pl.kernel,
    mesh=plsc.VectorSubcoreMesh(core_axis_name="c", subcore_axis_name="s",
                                num_cores=NC),
    out_type=jax.ShapeDtypeStruct((B, N), jnp.float32),
    scratch_types=[pltpu.VMEM((RW, N), jnp.float32),
                   pltpu.VMEM((RW, N), jnp.float32),
                   pltpu.SemaphoreType.DMA,
                   pltpu.SemaphoreType.DMA],
)
def _sc_gate(scores_hbm, gate_hbm, rows_v, gate_v, sem_in, sem_out):
    wid = lax.axis_index("s") * NC + lax.axis_index("c")
    base = wid * RW
    pltpu.sync_copy(scores_hbm.at[pl.ds(base, 1)], rows_v.at[pl.ds(0, 1)])
    in1 = pltpu.async_copy(scores_hbm.at[pl.ds(base + 1, 1)],
                           rows_v.at[pl.ds(1, 1)], sem_in)

    xor_idx = _xor_indices()
    neg = jnp.full((L,), -3.0e38, jnp.float32)
    out0 = None
    for r in range(RW):
        if r == 1:
            in1.wait()
        def chunk_body(j, lists):
            v = [rows_v[r, pl.ds((j * K + i) * L, L)] for i in range(K)]
            for a, b in ((0, 1), (2, 3), (4, 5), (6, 7),
                         (0, 2), (1, 3), (4, 6), (5, 7),
                         (1, 2), (5, 6), (0, 4), (3, 7),
                         (1, 5), (2, 6),
                         (1, 4), (3, 6),
                         (2, 4), (3, 5),
                         (3, 4)):
                hi = jnp.maximum(v[a], v[b])
                v[b] = jnp.minimum(v[a], v[b])
                v[a] = hi
            lists = list(lists)
            for k in range(K):
                w = v[k]
                for lv in range(k, K):
                    hi = jnp.maximum(lists[lv], w)
                    w = jnp.minimum(lists[lv], w)
                    lists[lv] = hi
            return tuple(lists)

        lists = lax.fori_loop(0, NCHUNK // K, chunk_body, (neg,) * K)

        cur = jnp.full((L,), 3.0e38, jnp.float32)
        remaining = jnp.full((L,), float(K), jnp.float32)
        thresh = neg
        for _ in range(K):
            mv = neg
            for li in lists:
                mv = jnp.maximum(mv, jnp.where(li < cur, li, neg))
            m = _allmax(mv, xor_idx)
            cnt = jnp.zeros((L,), jnp.float32)
            for li in lists:
                cnt = cnt + jnp.where(li == m, 1.0, 0.0)
            cnt = _allsum(cnt, xor_idx)
            tf = jnp.where(remaining > 0, jnp.float32(1.0), jnp.float32(0.0))
            thresh = tf * m + (1.0 - tf) * thresh
            remaining = remaining - tf * cnt
            cur = m

        def gate_body(j, carry):
            for i in range(K):
                off = (j * K + i) * L
                v = rows_v[r, pl.ds(off, L)]
                gate_v[r, pl.ds(off, L)] = jnp.where(
                    v >= thresh, jnp.float32(1.0), jnp.float32(0.0))
            return carry

        lax.fori_loop(0, NCHUNK // K, gate_body, 0)

        if r == 0:
            out0 = pltpu.async_copy(gate_v.at[pl.ds(0, 1)],
                                    gate_hbm.at[pl.ds(base, 1)], sem_out)

    pltpu.sync_copy(gate_v.at[pl.ds(1, 1)], gate_hbm.at[pl.ds(base + 1, 1)])
    out0.wait()


def kernel(periph_seq, imu_seq, traj_seq, W1, b1, W2, b2):
    scores = _tc_scores(periph_seq, imu_seq, traj_seq, W1, b1, W2, b2)
    gate = _sc_gate(scores)
    return (scores, gate)


# speedup vs baseline: 1.0558x; 1.0026x over previous
import functools

import jax
import jax.numpy as jnp
from jax import lax
from jax.experimental import pallas as pl
from jax.experimental.pallas import tpu as pltpu
from jax.experimental.pallas import tpu_sc as plsc

T, B = 32, 64
P, I, R = 1024, 64, 128
H, N = 64, 4096
K = 8

NC, NS, L = 2, 16, 16
NW = NC * NS
RW = B // NW
NCHUNK = N // L


TBLK = 16
NT = T // TBLK


def _scores_body(periph_ref, imu_ref, traj_ref, w1_ref, b1_ref, w2_ref,
                 b2_ref, scores_ref, acc_p, acc_i, acc_t):
    t = pl.program_id(0)

    @pl.when(t == 0)
    def _():
        acc_p[...] = jnp.zeros_like(acc_p)
        acc_i[...] = jnp.zeros_like(acc_i)
        acc_t[...] = jnp.zeros_like(acc_t)

    xp = acc_p[...]
    xi = acc_i[...]
    xt = acc_t[...]
    for i in range(TBLK):
        xp = xp + periph_ref[i]
        xi = xi + imu_ref[i]
        xt = xt + traj_ref[i]
    acc_p[...] = xp
    acc_i[...] = xi
    acc_t[...] = xt

    @pl.when(t == NT - 1)
    def _():
        inv = jnp.float32(1.0 / T)
        xp = acc_p[...] * inv
        xi = acc_i[...] * inv
        xt = acc_t[...] * inv
        w1 = w1_ref[...]
        h = (jnp.dot(xp, w1[0:P], preferred_element_type=jnp.float32)
             + jnp.dot(xi, w1[P:P + I], preferred_element_type=jnp.float32)
             + jnp.dot(xt, w1[P + I:P + I + R],
                       preferred_element_type=jnp.float32)
             + b1_ref[...])
        h = jnp.maximum(h, 0.0)
        scores_ref[...] = (
            jnp.dot(h, w2_ref[...], preferred_element_type=jnp.float32)
            + b2_ref[...])


def _tc_scores(periph_seq, imu_seq, traj_seq, W1, b1, W2, b2):
    return pl.pallas_call(
        _scores_body,
        grid=(NT,),
        in_specs=[
            pl.BlockSpec((TBLK, B, P), lambda t: (t, 0, 0)),
            pl.BlockSpec((TBLK, B, I), lambda t: (t, 0, 0)),
            pl.BlockSpec((TBLK, B, R), lambda t: (t, 0, 0)),
            pl.BlockSpec((P + I + R, H), lambda t: (0, 0)),
            pl.BlockSpec((1, H), lambda t: (0, 0)),
            pl.BlockSpec((H, N), lambda t: (0, 0)),
            pl.BlockSpec((1, N), lambda t: (0, 0)),
        ],
        out_specs=pl.BlockSpec((B, N), lambda t: (0, 0)),
        out_shape=jax.ShapeDtypeStruct((B, N), jnp.float32),
        scratch_shapes=[pltpu.VMEM((B, P), jnp.float32),
                        pltpu.VMEM((B, I), jnp.float32),
                        pltpu.VMEM((B, R), jnp.float32)],
    )(periph_seq, imu_seq, traj_seq, W1, b1.reshape(1, H), W2,
      b2.reshape(1, N))


_GATHER_DNUMS = lax.GatherDimensionNumbers(
    offset_dims=(), collapsed_slice_dims=(0,), start_index_map=(0,))


def _shuffle(x, idx):
    return lax.gather(x, idx, _GATHER_DNUMS, slice_sizes=(1,),
                      mode=lax.GatherScatterMode.PROMISE_IN_BOUNDS)


def _xor_indices():
    iota = lax.iota(jnp.int32, L)
    return [jnp.reshape(jnp.bitwise_xor(iota, s), (L, 1))
            for s in (1, 2, 4, 8)]


def _allmax(x, xor_idx):
    for idx in xor_idx:
        x = jnp.maximum(x, _shuffle(x, idx))
    return x


def _allsum(x, xor_idx):
    for idx in xor_idx:
        x = x + _shuffle(x, idx)
    return x


@functools.partial(
    pl.kernel,
    mesh=plsc.VectorSubcoreMesh(core_axis_name="c", subcore_axis_name="s",
                                num_cores=NC),
    out_type=jax.ShapeDtypeStruct((B, N), jnp.float32),
    scratch_types=[pltpu.VMEM((RW, N), jnp.float32),
                   pltpu.VMEM((RW, N), jnp.float32)],
)
def _sc_gate(scores_hbm, gate_hbm, rows_v, gate_v):
    wid = lax.axis_index("s") * NC + lax.axis_index("c")
    base = wid * RW
    pltpu.sync_copy(scores_hbm.at[pl.ds(base, RW)], rows_v)

    xor_idx = _xor_indices()
    neg = jnp.full((L,), -3.0e38, jnp.float32)
    for r in range(RW):
        def chunk_body(j, lists):
            v = [rows_v[r, pl.ds((j * K + i) * L, L)] for i in range(K)]
            for a, b in ((0, 1), (2, 3), (4, 5), (6, 7),
                         (0, 2), (1, 3), (4, 6), (5, 7),
                         (1, 2), (5, 6), (0, 4), (3, 7),
                         (1, 5), (2, 6),
                         (1, 4), (3, 6),
                         (2, 4), (3, 5),
                         (3, 4)):
                hi = jnp.maximum(v[a], v[b])
                v[b] = jnp.minimum(v[a], v[b])
                v[a] = hi
            lists = list(lists)
            for k in range(K):
                w = v[k]
                for lv in range(k, K):
                    hi = jnp.maximum(lists[lv], w)
                    w = jnp.minimum(lists[lv], w)
                    lists[lv] = hi
            return tuple(lists)

        lists = lax.fori_loop(0, NCHUNK // K, chunk_body, (neg,) * K)

        cur = jnp.full((L,), 3.0e38, jnp.float32)
        remaining = jnp.full((L,), float(K), jnp.float32)
        thresh = neg
        for _ in range(K):
            mv = neg
            for li in lists:
                mv = jnp.maximum(mv, jnp.where(li < cur, li, neg))
            m = _allmax(mv, xor_idx)
            cnt = jnp.zeros((L,), jnp.float32)
            for li in lists:
                cnt = cnt + jnp.where(li == m, 1.0, 0.0)
            cnt = _allsum(cnt, xor_idx)
            tf = jnp.where(remaining > 0, jnp.float32(1.0), jnp.float32(0.0))
            thresh = tf * m + (1.0 - tf) * thresh
            remaining = remaining - tf * cnt
            cur = m

        def gate_body(j, carry):
            for i in range(K):
                off = (j * K + i) * L
                v = rows_v[r, pl.ds(off, L)]
                gate_v[r, pl.ds(off, L)] = jnp.where(
                    v >= thresh, jnp.float32(1.0), jnp.float32(0.0))
            return carry

        lax.fori_loop(0, NCHUNK // K, gate_body, 0)

    pltpu.sync_copy(gate_v, gate_hbm.at[pl.ds(base, RW)])


def kernel(periph_seq, imu_seq, traj_seq, W1, b1, W2, b2):
    scores = _tc_scores(periph_seq, imu_seq, traj_seq, W1, b1, W2, b2)
    gate = _sc_gate(scores)
    return (scores, gate)
